# Initial kernel scaffold; baseline (speedup 1.0000x reference)
import jax
import jax.numpy as jnp
from jax.experimental import pallas as pl


def kernel(z, edge_index, W1, b1, W2, b2, W3, b3):
    raise NotImplementedError("write your pallas kernel here")


# speedup vs baseline: 6.9385x; 6.9385x over previous
import functools

import jax
import jax.numpy as jnp
from jax import lax
from jax.experimental import pallas as pl
from jax.experimental.pallas import tpu as pltpu
from jax.experimental.pallas import tpu_sc as plsc

N_NODES = 10000
D = 128
E_EDGES = 320000

NPAD = 10240
EPAD = 327680
GROUPS = EPAD // (32 * 128)
ROWS_PER_TILE = NPAD // 16
HIST_W = 16

_mesh = plsc.VectorSubcoreMesh(core_axis_name="c", subcore_axis_name="s")


def _zero_vmem(ref, nrows, ncols):
  def body(i, _):
    r = i // (ncols // 16)
    l = (i % (ncols // 16)) * 16
    ref[r, pl.ds(l, 16)] = jnp.zeros((16,), jnp.float32)
    return 0
  lax.fori_loop(0, nrows * (ncols // 16), body, 0)


@functools.partial(
    pl.kernel,
    out_type=jax.ShapeDtypeStruct((2 * NPAD, HIST_W), jnp.float32),
    mesh=_mesh,
    scratch_types=[
        pltpu.VMEM((GROUPS, 128), jnp.int32),
        pltpu.VMEM((128, HIST_W), jnp.float32),
        pltpu.VMEM((128, HIST_W), jnp.float32),
        pltpu.VMEM_SHARED((NPAD, HIST_W), jnp.float32),
    ],
)
def _sc_degree(dst_hbm, out_hbm, idx_d, ones_v, stage_v, acc):
  c = lax.axis_index("c")
  s = lax.axis_index("s")
  wid = s * 2 + c

  def fill(i, _):
    ones_v[i, pl.ds(0, 16)] = jnp.ones((16,), jnp.float32)
    stage_v[i, pl.ds(0, 16)] = jnp.zeros((16,), jnp.float32)
    return 0
  lax.fori_loop(0, 128, fill, 0)

  base = s * ROWS_PER_TILE
  for k in range(ROWS_PER_TILE // 128):
    pltpu.sync_copy(stage_v, acc.at[pl.ds(base + k * 128, 128)])
  plsc.subcore_barrier()

  pltpu.sync_copy(dst_hbm.at[pl.ds(wid * GROUPS, GROUPS)], idx_d)

  def group(j, _):
    pltpu.sync_copy(ones_v, acc.at[idx_d.at[j]], add=True)
    return 0
  lax.fori_loop(0, GROUPS, group, 0)
  plsc.subcore_barrier()

  out_base = c * NPAD + base
  for k in range(ROWS_PER_TILE // 128):
    pltpu.sync_copy(acc.at[pl.ds(base + k * 128, 128)], stage_v)
    pltpu.sync_copy(stage_v, out_hbm.at[pl.ds(out_base + k * 128, 128)])


@functools.partial(
    pl.kernel,
    out_type=jax.ShapeDtypeStruct((2 * NPAD, D), jnp.float32),
    mesh=_mesh,
    scratch_types=[
        pltpu.VMEM((GROUPS, 128), jnp.int32),
        pltpu.VMEM((GROUPS, 128), jnp.int32),
        pltpu.VMEM((128, D), jnp.float32),
        pltpu.VMEM_SHARED((NPAD, D), jnp.float32),
        pltpu.SemaphoreType.DMA,
    ],
)
def _sc_aggregate(y_hbm, src_hbm, dst_hbm, out_hbm, idx_s, idx_d, rows_v, acc, sem):
  c = lax.axis_index("c")
  s = lax.axis_index("s")
  wid = s * 2 + c

  _zero_vmem(rows_v, 128, D)
  base = s * ROWS_PER_TILE
  for k in range(ROWS_PER_TILE // 128):
    pltpu.sync_copy(rows_v, acc.at[pl.ds(base + k * 128, 128)])
  plsc.subcore_barrier()

  pltpu.sync_copy(src_hbm.at[pl.ds(wid * GROUPS, GROUPS)], idx_s)
  pltpu.sync_copy(dst_hbm.at[pl.ds(wid * GROUPS, GROUPS)], idx_d)

  def group(j, _):
    pltpu.async_copy(y_hbm.at[idx_s.at[j]], rows_v, sem).wait()
    pltpu.sync_copy(rows_v, acc.at[idx_d.at[j]], add=True)
    return 0
  lax.fori_loop(0, GROUPS, group, 0)
  plsc.subcore_barrier()

  out_base = c * NPAD + base
  for k in range(ROWS_PER_TILE // 128):
    pltpu.sync_copy(acc.at[pl.ds(base + k * 128, 128)], rows_v)
    pltpu.sync_copy(rows_v, out_hbm.at[pl.ds(out_base + k * 128, 128)])


_RB = 1024


def _tc_pre(z, W1, h0, h1):
  grid = (NPAD // _RB,)

  def body(z_ref, w_ref, h0_ref, h1_ref, dinv_ref, y_ref):
    g = pl.program_id(0)
    deg = h0_ref[:, 0:1] + h1_ref[:, 0:1] + 1.0
    dinv = lax.rsqrt(deg)
    rows = g * _RB + lax.broadcasted_iota(jnp.int32, (_RB, 1), 0)
    dinv = jnp.where(rows < N_NODES, dinv, 0.0)
    dinv_b = jnp.broadcast_to(dinv, (_RB, D))
    dinv_ref[...] = dinv_b
    y_ref[...] = dinv_b * jnp.dot(z_ref[...], w_ref[...],
                                  preferred_element_type=jnp.float32)

  return pl.pallas_call(
      body,
      grid=grid,
      in_specs=[
          pl.BlockSpec((_RB, D), lambda g: (g, 0)),
          pl.BlockSpec((D, D), lambda g: (0, 0)),
          pl.BlockSpec((_RB, HIST_W), lambda g: (g, 0)),
          pl.BlockSpec((_RB, HIST_W), lambda g: (g, 0)),
      ],
      out_specs=[
          pl.BlockSpec((_RB, D), lambda g: (g, 0)),
          pl.BlockSpec((_RB, D), lambda g: (g, 0)),
      ],
      out_shape=[
          jax.ShapeDtypeStruct((NPAD, D), jnp.float32),
          jax.ShapeDtypeStruct((NPAD, D), jnp.float32),
      ],
  )(z, W1, h0, h1)


def _tc_mid(a0, a1, y, dinv, b, W):
  grid = (NPAD // _RB,)

  def body(a0_ref, a1_ref, y_ref, dinv_ref, b_ref, w_ref, out_ref):
    dv = dinv_ref[...]
    h = dv * (a0_ref[...] + a1_ref[...] + y_ref[...]) + b_ref[...]
    h = jnp.maximum(h, 0.0)
    out_ref[...] = dv * jnp.dot(h, w_ref[...], preferred_element_type=jnp.float32)

  return pl.pallas_call(
      body,
      grid=grid,
      in_specs=[
          pl.BlockSpec((_RB, D), lambda g: (g, 0)),
          pl.BlockSpec((_RB, D), lambda g: (g, 0)),
          pl.BlockSpec((_RB, D), lambda g: (g, 0)),
          pl.BlockSpec((_RB, D), lambda g: (g, 0)),
          pl.BlockSpec((1, D), lambda g: (0, 0)),
          pl.BlockSpec((D, D), lambda g: (0, 0)),
      ],
      out_specs=pl.BlockSpec((_RB, D), lambda g: (g, 0)),
      out_shape=jax.ShapeDtypeStruct((NPAD, D), jnp.float32),
  )(a0, a1, y, dinv, b, W)


def _tc_post(a0, a1, y, dinv, b):
  grid = (NPAD // _RB,)

  def body(a0_ref, a1_ref, y_ref, dinv_ref, b_ref, out_ref):
    out_ref[...] = (dinv_ref[...] * (a0_ref[...] + a1_ref[...] + y_ref[...])
                    + b_ref[...])

  return pl.pallas_call(
      body,
      grid=grid,
      in_specs=[
          pl.BlockSpec((_RB, D), lambda g: (g, 0)),
          pl.BlockSpec((_RB, D), lambda g: (g, 0)),
          pl.BlockSpec((_RB, D), lambda g: (g, 0)),
          pl.BlockSpec((_RB, D), lambda g: (g, 0)),
          pl.BlockSpec((1, D), lambda g: (0, 0)),
      ],
      out_specs=pl.BlockSpec((_RB, D), lambda g: (g, 0)),
      out_shape=jax.ShapeDtypeStruct((NPAD, D), jnp.float32),
  )(a0, a1, y, dinv, b)


@jax.jit
def kernel(z, edge_index, W1, b1, W2, b2, W3, b3):
  src = edge_index[0]
  dst = edge_index[1]
  pad = EPAD - E_EDGES
  src2d = jnp.concatenate(
      [src, jnp.full((pad,), N_NODES, jnp.int32)]).reshape(EPAD // 128, 128)
  dst2d = jnp.concatenate(
      [dst, jnp.full((pad,), N_NODES, jnp.int32)]).reshape(EPAD // 128, 128)
  z_pad = jnp.pad(z, ((0, NPAD - N_NODES), (0, 0)))
  b1r = b1.reshape(1, D)
  b2r = b2.reshape(1, D)
  b3r = b3.reshape(1, D)

  hist = _sc_degree(dst2d)
  h0 = hist[:NPAD]
  h1 = hist[NPAD:]
  dinv, y = _tc_pre(z_pad, W1, h0, h1)

  agg = _sc_aggregate(y, src2d, dst2d)
  y = _tc_mid(agg[:NPAD], agg[NPAD:], y, dinv, b1r, W2)

  agg = _sc_aggregate(y, src2d, dst2d)
  y = _tc_mid(agg[:NPAD], agg[NPAD:], y, dinv, b2r, W3)

  agg = _sc_aggregate(y, src2d, dst2d)
  out = _tc_post(agg[:NPAD], agg[NPAD:], y, dinv, b3r)
  return out[:N_NODES]
